# baseline (speedup 1.0000x reference)
import functools

import jax
import jax.numpy as jnp
from jax import lax
from jax.experimental import pallas as pl
from jax.experimental.pallas import tpu as pltpu
from jax.experimental.pallas import tpu_sc as plsc

N = 34200
E = 547200
NPAD = 34304
EPAD = 589824
NROWS = EPAD // 128
TPN = NPAD // 16
CW = 16
NCH1 = 128 // CW
NCH2 = 64 // CW
DEGW = 16
BN = 256
NBLK = NPAD // BN


def _sc_mesh():
    return plsc.VectorSubcoreMesh(core_axis_name="c", subcore_axis_name="s",
                                  num_cores=2, num_subcores=16)


@functools.cache
def _make_deg():
    @functools.partial(
        pl.kernel,
        out_type=jax.ShapeDtypeStruct((2, NPAD, DEGW), jnp.float32),
        mesh=_sc_mesh(),
        compiler_params=pltpu.CompilerParams(use_tc_tiling_on_sc=False),
        scratch_types=[
            pltpu.VMEM_SHARED((NPAD, DEGW), jnp.float32),
            pltpu.VMEM((TPN, DEGW), jnp.float32),
            pltpu.VMEM((128, DEGW), jnp.float32),
            pltpu.VMEM((128,), jnp.int32),
        ],
    )
    def _deg(dst_hbm, ones_hbm, zer_hbm, out_hbm, acc, zbuf, ones_v, dst_v):
        c = lax.axis_index("c")
        s = lax.axis_index("s")
        pltpu.sync_copy(zer_hbm, zbuf)
        pltpu.sync_copy(ones_hbm, ones_v)
        pltpu.sync_copy(zbuf, acc.at[pl.ds(s * TPN, TPN)])
        plsc.subcore_barrier()
        rows_per_tile = NROWS // 2 // 16
        base = (c * (NROWS // 2) + s * rows_per_tile) * 128

        def body(r, carry):
            pltpu.sync_copy(dst_hbm.at[pl.ds(base + r * 128, 128)], dst_v)
            pltpu.sync_copy(ones_v, acc.at[dst_v], add=True)
            return carry

        lax.fori_loop(0, rows_per_tile, body, 0)
        plsc.subcore_barrier()
        pltpu.sync_copy(acc.at[pl.ds(s * TPN, TPN)],
                        out_hbm.at[c, pl.ds(s * TPN, TPN)])

    return _deg


def _deg_sc(dstr, ones16, zer16):
    return _make_deg()(dstr, ones16, zer16)


RB = 8
RPT = NROWS // 16
NB = RPT // RB


@functools.cache
def _make_scatter(n_chunks):
    cpc = n_chunks // 2

    @functools.partial(
        pl.kernel,
        out_type=jax.ShapeDtypeStruct((n_chunks, NPAD, CW), jnp.float32),
        mesh=_sc_mesh(),
        compiler_params=pltpu.CompilerParams(use_tc_tiling_on_sc=False),
        scratch_types=[
            pltpu.VMEM_SHARED((NPAD, CW), jnp.float32),
            pltpu.VMEM((TPN, CW), jnp.float32),
            pltpu.VMEM((3, RB * 128), jnp.int32),
            pltpu.VMEM((3, RB * 128), jnp.int32),
            pltpu.VMEM((3, RB * 128, CW), jnp.float32),
            pltpu.SemaphoreType.DMA,
            pltpu.SemaphoreType.DMA,
            pltpu.SemaphoreType.DMA,
            pltpu.SemaphoreType.DMA,
            pltpu.SemaphoreType.DMA,
            pltpu.SemaphoreType.DMA,
        ],
    )
    def _scat(y_hbm, src_hbm, dst_hbm, zer_hbm, out_hbm,
              acc, zbuf, src_v, dst_v, vals,
              g0, g1, g2, s0, s1, s2):
        c = lax.axis_index("c")
        s = lax.axis_index("s")
        gsem = (g0, g1, g2)
        ssem = (s0, s1, s2)
        pltpu.sync_copy(zer_hbm, zbuf)
        EB = RB * 128
        base = s * RPT * 128

        for p in range(cpc):
            ch = c * cpc + p
            tbl = y_hbm.at[ch]
            pltpu.sync_copy(zbuf, acc.at[pl.ds(s * TPN, TPN)])
            plsc.subcore_barrier()

            def load_idx(t, slot):
                pltpu.sync_copy(src_hbm.at[pl.ds(base + t * EB, EB)],
                                src_v.at[slot])
                pltpu.sync_copy(dst_hbm.at[pl.ds(base + t * EB, EB)],
                                dst_v.at[slot])

            def fire_g(slot):
                pltpu.async_copy(tbl.at[src_v.at[slot]],
                                 vals.at[slot], gsem[slot])

            def wait_g(slot):
                pltpu.make_async_copy(tbl.at[src_v.at[slot]],
                                      vals.at[slot], gsem[slot]).wait()

            def scat_sync(slot):
                pltpu.sync_copy(vals.at[slot],
                                acc.at[dst_v.at[slot]], add=True)

            def body(t, carry):
                load_idx(t, 0)
                fire_g(0)
                wait_g(0)
                scat_sync(0)
                return carry

            lax.fori_loop(0, NB, body, 0)

            plsc.subcore_barrier()
            pltpu.sync_copy(acc.at[pl.ds(s * TPN, TPN)],
                            out_hbm.at[ch, pl.ds(s * TPN, TPN)])
            plsc.subcore_barrier()

    return _scat


def _scat4(y1c, srcr, dstr, zerc):
    return _make_scatter(NCH1)(y1c, srcr, dstr, zerc)


def _scat2(y2c, srcr, dstr, zerc):
    return _make_scatter(NCH2)(y2c, srcr, dstr, zerc)


def _dinv_from(deg_blk):
    deg = deg_blk[0, :, 0:1] + deg_blk[1, :, 0:1] + 1.0
    return lax.rsqrt(deg)


def _mm1(xp, W1):
    def body(x_ref, w_ref, o_ref):
        o_ref[...] = jnp.dot(x_ref[...], w_ref[...],
                             preferred_element_type=jnp.float32)

    return pl.pallas_call(
        body,
        grid=(NBLK,),
        in_specs=[pl.BlockSpec((BN, 128), lambda i: (i, 0)),
                  pl.BlockSpec((128, 128), lambda i: (0, 0))],
        out_specs=pl.BlockSpec((BN, 128), lambda i: (i, 0)),
        out_shape=jax.ShapeDtypeStruct((NPAD, 128), jnp.float32),
    )(xp, W1)


def _ychunk(xw, degp):
    def body(x_ref, d_ref, o_ref):
        y = _dinv_from(d_ref[...]) * x_ref[...]
        for c in range(NCH1):
            o_ref[c] = y[:, CW * c:CW * (c + 1)]

    return pl.pallas_call(
        body,
        grid=(NBLK,),
        in_specs=[pl.BlockSpec((BN, 128), lambda i: (i, 0)),
                  pl.BlockSpec((2, BN, DEGW), lambda i: (0, i, 0))],
        out_specs=pl.BlockSpec((NCH1, BN, CW), lambda i: (0, i, 0)),
        out_shape=jax.ShapeDtypeStruct((NCH1, NPAD, CW), jnp.float32),
    )(xw, degp)


def _stat(sseg, yc, degp, br, nch):
    def body(s_ref, y_ref, d_ref, b_ref, h_ref, sum_ref):
        i = pl.program_id(0)
        dinv = _dinv_from(d_ref[...])
        bb = b_ref[...]
        rows = lax.broadcasted_iota(jnp.int32, (BN, 1), 0) + i * BN
        mask = rows < N
        parts = []
        for c in range(nch):
            h = dinv * (s_ref[c] + y_ref[c]) + bb[c:c + 1]
            e = jnp.where(h > 0, h, jnp.exp(h) - 1.0)
            h_ref[c] = e
            em = jnp.where(mask, e, 0.0)
            parts.append(jnp.concatenate(
                [jnp.sum(em, axis=0, keepdims=True),
                 jnp.sum(em * em, axis=0, keepdims=True)], axis=0)[None])

        @pl.when(i == 0)
        def _():
            sum_ref[...] = jnp.zeros_like(sum_ref)

        sum_ref[...] += jnp.concatenate(parts, axis=0)

    return pl.pallas_call(
        body,
        grid=(NBLK,),
        in_specs=[pl.BlockSpec((nch, BN, CW), lambda i: (0, i, 0)),
                  pl.BlockSpec((nch, BN, CW), lambda i: (0, i, 0)),
                  pl.BlockSpec((2, BN, DEGW), lambda i: (0, i, 0)),
                  pl.BlockSpec((nch, CW), lambda i: (0, 0))],
        out_specs=[pl.BlockSpec((nch, BN, CW), lambda i: (0, i, 0)),
                   pl.BlockSpec((nch, 2, CW), lambda i: (0, 0, 0))],
        out_shape=[jax.ShapeDtypeStruct((nch, NPAD, CW), jnp.float32),
                   jax.ShapeDtypeStruct((nch, 2, CW), jnp.float32)],
    )(sseg, yc, degp, br)


def _gnorm_cols(hc, m, ms, w, b, eh2):
    var = eh2 - m * m * ms * (2.0 - ms)
    return (hc - m * ms) * (w * lax.rsqrt(var + 1e-5)) + b


def _mm2(h1c, sums, degp, gw, gb, gms, W2r):
    def body(h_ref, sm_ref, d_ref, gw_ref, gb_ref, gms_ref, w_ref, o_ref):
        dinv = _dinv_from(d_ref[...])
        h = h_ref[...]
        sm = sm_ref[...]
        g_w = gw_ref[...]
        g_b = gb_ref[...]
        g_ms = gms_ref[...]
        w = w_ref[...]
        acc = jnp.zeros((BN, 64), jnp.float32)
        for c in range(NCH1):
            m = sm[c, 0:1, :] * (1.0 / N)
            eh2 = sm[c, 1:2, :] * (1.0 / N)
            gc = _gnorm_cols(h[c], m, g_ms[c:c + 1], g_w[c:c + 1],
                             g_b[c:c + 1], eh2)
            acc = acc + jnp.dot(gc, w[c], preferred_element_type=jnp.float32)
        y2 = dinv * acc
        for c in range(NCH2):
            o_ref[c] = y2[:, CW * c:CW * (c + 1)]

    return pl.pallas_call(
        body,
        grid=(NBLK,),
        in_specs=[pl.BlockSpec((NCH1, BN, CW), lambda i: (0, i, 0)),
                  pl.BlockSpec((NCH1, 2, CW), lambda i: (0, 0, 0)),
                  pl.BlockSpec((2, BN, DEGW), lambda i: (0, i, 0)),
                  pl.BlockSpec((NCH1, CW), lambda i: (0, 0)),
                  pl.BlockSpec((NCH1, CW), lambda i: (0, 0)),
                  pl.BlockSpec((NCH1, CW), lambda i: (0, 0)),
                  pl.BlockSpec((NCH1, CW, 64), lambda i: (0, 0, 0))],
        out_specs=pl.BlockSpec((NCH2, BN, CW), lambda i: (0, i, 0)),
        out_shape=jax.ShapeDtypeStruct((NCH2, NPAD, CW), jnp.float32),
    )(h1c, sums, degp, gw, gb, gms, W2r)


def _norm(h2c, sums, gw, gb, gms, nch):
    def body(h_ref, sm_ref, gw_ref, gb_ref, gms_ref, o_ref):
        sm = sm_ref[...]
        g_w = gw_ref[...]
        g_b = gb_ref[...]
        g_ms = gms_ref[...]
        for c in range(nch):
            m = sm[c, 0:1, :] * (1.0 / N)
            eh2 = sm[c, 1:2, :] * (1.0 / N)
            o_ref[c] = _gnorm_cols(h_ref[c], m, g_ms[c:c + 1], g_w[c:c + 1],
                                   g_b[c:c + 1], eh2)

    return pl.pallas_call(
        body,
        grid=(NBLK,),
        in_specs=[pl.BlockSpec((nch, BN, CW), lambda i: (0, i, 0)),
                  pl.BlockSpec((nch, 2, CW), lambda i: (0, 0, 0)),
                  pl.BlockSpec((nch, CW), lambda i: (0, 0)),
                  pl.BlockSpec((nch, CW), lambda i: (0, 0)),
                  pl.BlockSpec((nch, CW), lambda i: (0, 0))],
        out_specs=pl.BlockSpec((nch, BN, CW), lambda i: (0, i, 0)),
        out_shape=jax.ShapeDtypeStruct((nch, NPAD, CW), jnp.float32),
    )(h2c, sums, gw, gb, gms)


def _head(A, Wstack, b1r, bng, bnb, bnm, bnv, W2h, b2r):
    KA = 228 * CW

    def body(a_ref, w_ref, b1_ref, g_ref, bb_ref, m_ref, v_ref,
             w2_ref, b2_ref, o_ref, acc_ref):
        c = pl.program_id(0)

        @pl.when(c == 0)
        def _():
            acc_ref[...] = jnp.zeros_like(acc_ref)

        acc_ref[...] += jnp.dot(a_ref[0], w_ref[0],
                                preferred_element_type=jnp.float32)

        @pl.when(c == NCH2 - 1)
        def _():
            z = acc_ref[...] + b1_ref[...]
            z = jnp.where(z > 0, z, jnp.exp(z) - 1.0)
            z = (z - m_ref[...]) * (g_ref[...] * lax.rsqrt(v_ref[...] + 1e-5)) \
                + bb_ref[...]
            o_ref[...] = jnp.dot(z, w2_ref[...],
                                 preferred_element_type=jnp.float32) \
                + b2_ref[...]

    return pl.pallas_call(
        body,
        grid=(NCH2,),
        in_specs=[pl.BlockSpec((1, 150, KA), lambda c: (c, 0, 0)),
                  pl.BlockSpec((1, KA, 128), lambda c: (c, 0, 0)),
                  pl.BlockSpec((1, 128), lambda c: (0, 0)),
                  pl.BlockSpec((1, 128), lambda c: (0, 0)),
                  pl.BlockSpec((1, 128), lambda c: (0, 0)),
                  pl.BlockSpec((1, 128), lambda c: (0, 0)),
                  pl.BlockSpec((1, 128), lambda c: (0, 0)),
                  pl.BlockSpec((128, 10), lambda c: (0, 0)),
                  pl.BlockSpec((1, 10), lambda c: (0, 0))],
        out_specs=pl.BlockSpec((150, 10), lambda c: (0, 0)),
        out_shape=jax.ShapeDtypeStruct((150, 10), jnp.float32),
        scratch_shapes=[pltpu.VMEM((150, 128), jnp.float32)],
    )(A, Wstack, b1r, bng, bnb, bnm, bnv, W2h, b2r)


def kernel(x, edge_index, W1, b1, W2, b2, gn1_w, gn1_b, gn1_ms,
           gn2_w, gn2_b, gn2_ms, lin1_W, lin1_b, bn_g, bn_b, bn_m, bn_v,
           lin2_W, lin2_b):
    src = edge_index[0].astype(jnp.int32)
    dst = edge_index[1].astype(jnp.int32)
    pad_e = EPAD - E
    pad_dst = N + jnp.arange(pad_e, dtype=jnp.int32) % (NPAD - N)
    srcr = jnp.concatenate([src, jnp.zeros((pad_e,), jnp.int32)])
    dstr = jnp.concatenate([dst, pad_dst])
    xp = jnp.pad(x, ((0, NPAD - N), (0, 0)))
    ones16 = jnp.ones((128, DEGW), jnp.float32)
    zer16 = jnp.zeros((TPN, DEGW), jnp.float32)
    zerc = jnp.zeros((TPN, CW), jnp.float32)

    degp = _deg_sc(dstr, ones16, zer16)
    xw = _mm1(xp, W1)
    y1c = _ychunk(xw, degp)
    s1 = _scat4(y1c, srcr, dstr, zerc)
    h1c, sums1 = _stat(s1, y1c, degp, b1.reshape(NCH1, CW), NCH1)
    y2c = _mm2(h1c, sums1, degp, gn1_w.reshape(NCH1, CW),
               gn1_b.reshape(NCH1, CW), gn1_ms.reshape(NCH1, CW),
               W2.reshape(NCH1, CW, 64))
    s2 = _scat2(y2c, srcr, dstr, zerc)
    h2c, sums2 = _stat(s2, y2c, degp, b2.reshape(NCH2, CW), NCH2)
    g2c = _norm(h2c, sums2, gn2_w.reshape(NCH2, CW), gn2_b.reshape(NCH2, CW),
                gn2_ms.reshape(NCH2, CW), NCH2)
    A = g2c[:, :N, :].reshape(NCH2, 150, 228 * CW)
    Wstack = lin1_W.reshape(228, NCH2, CW, 128).transpose(1, 0, 2, 3) \
        .reshape(NCH2, 228 * CW, 128)
    return _head(A, Wstack, lin1_b.reshape(1, 128), bn_g.reshape(1, 128),
                 bn_b.reshape(1, 128), bn_m.reshape(1, 128),
                 bn_v.reshape(1, 128), lin2_W, lin2_b.reshape(1, 10))


# speedup vs baseline: 1.8230x; 1.8230x over previous
import functools

import jax
import jax.numpy as jnp
from jax import lax
from jax.experimental import pallas as pl
from jax.experimental.pallas import tpu as pltpu
from jax.experimental.pallas import tpu_sc as plsc

N = 34200
E = 547200
NPAD = 34304
EPAD = 548864
NROWS = EPAD // 128
TPN = NPAD // 16
CW = 16
NCH1 = 128 // CW
NCH2 = 64 // CW
DEGW = 16
BN = 256
NBLK = NPAD // BN


def _sc_mesh():
    return plsc.VectorSubcoreMesh(core_axis_name="c", subcore_axis_name="s",
                                  num_cores=2, num_subcores=16)


@functools.cache
def _make_deg():
    @functools.partial(
        pl.kernel,
        out_type=jax.ShapeDtypeStruct((2, NPAD, DEGW), jnp.float32),
        mesh=_sc_mesh(),
        compiler_params=pltpu.CompilerParams(use_tc_tiling_on_sc=False),
        scratch_types=[
            pltpu.VMEM_SHARED((NPAD, DEGW), jnp.float32),
            pltpu.VMEM((TPN, DEGW), jnp.float32),
            pltpu.VMEM((128, DEGW), jnp.float32),
            pltpu.VMEM((128,), jnp.int32),
        ],
    )
    def _deg(dst_hbm, ones_hbm, zer_hbm, out_hbm, acc, zbuf, ones_v, dst_v):
        c = lax.axis_index("c")
        s = lax.axis_index("s")
        pltpu.sync_copy(zer_hbm, zbuf)
        pltpu.sync_copy(ones_hbm, ones_v)
        pltpu.sync_copy(zbuf, acc.at[pl.ds(s * TPN, TPN)])
        plsc.subcore_barrier()
        rows_per_tile = NROWS // 2 // 16
        base = (c * (NROWS // 2) + s * rows_per_tile) * 128

        def body(r, carry):
            pltpu.sync_copy(dst_hbm.at[pl.ds(base + r * 128, 128)], dst_v)
            pltpu.sync_copy(ones_v, acc.at[dst_v], add=True)
            return carry

        lax.fori_loop(0, rows_per_tile, body, 0)
        plsc.subcore_barrier()
        pltpu.sync_copy(acc.at[pl.ds(s * TPN, TPN)],
                        out_hbm.at[c, pl.ds(s * TPN, TPN)])

    return _deg


def _deg_sc(dstr, ones16, zer16):
    return _make_deg()(dstr, ones16, zer16)


RB = 4
RPT = NROWS // 16
NB = RPT // RB


@functools.cache
def _make_scatter(n_chunks):
    cpc = n_chunks // 2

    @functools.partial(
        pl.kernel,
        out_type=jax.ShapeDtypeStruct((n_chunks, NPAD, CW), jnp.float32),
        mesh=_sc_mesh(),
        compiler_params=pltpu.CompilerParams(use_tc_tiling_on_sc=False),
        scratch_types=[
            pltpu.VMEM_SHARED((NPAD, CW), jnp.float32),
            pltpu.VMEM((TPN, CW), jnp.float32),
            pltpu.VMEM((2, RB, 128), jnp.int32),
            pltpu.VMEM((2, RB, 128), jnp.int32),
            pltpu.VMEM((2, RB, 128, CW), jnp.float32),
            pltpu.SemaphoreType.DMA,
            pltpu.SemaphoreType.DMA,
        ],
    )
    def _scat(y_hbm, src2d_hbm, dst2d_hbm, zer_hbm, out_hbm,
              acc, zbuf, src_v, dst_v, vals, g0, g1):
        c = lax.axis_index("c")
        s = lax.axis_index("s")
        gsem = (g0, g1)
        pltpu.sync_copy(zer_hbm, zbuf)
        base = s * RPT

        for p in range(cpc):
            ch = c * cpc + p
            tbl = y_hbm.at[ch]
            pltpu.sync_copy(zbuf, acc.at[pl.ds(s * TPN, TPN)])
            plsc.subcore_barrier()

            def load_idx(t, slot):
                pltpu.sync_copy(src2d_hbm.at[pl.ds(base + t * RB, RB)],
                                src_v.at[slot])
                pltpu.sync_copy(dst2d_hbm.at[pl.ds(base + t * RB, RB)],
                                dst_v.at[slot])

            def fire_g(slot):
                for j in range(RB):
                    pltpu.async_copy(tbl.at[src_v.at[slot, j]],
                                     vals.at[slot, j], gsem[slot])

            def wait_g(slot):
                for j in range(RB):
                    pltpu.make_async_copy(tbl.at[src_v.at[slot, j]],
                                          vals.at[slot, j], gsem[slot]).wait()

            def scat_sync(slot):
                for j in range(RB):
                    pltpu.sync_copy(vals.at[slot, j],
                                    acc.at[dst_v.at[slot, j]], add=True)

            def block(t, slot, prefetch):
                if prefetch:
                    load_idx(t + 1, 1 - slot)
                    fire_g(1 - slot)
                wait_g(slot)
                scat_sync(slot)

            load_idx(0, 0)
            fire_g(0)

            def body(t2, carry):
                t = 2 * t2
                block(t, 0, True)
                block(t + 1, 1, True)
                return carry

            lax.fori_loop(0, (NB - 1) // 2, body, 0)
            block(NB - 1, 0, False)

            plsc.subcore_barrier()
            pltpu.sync_copy(acc.at[pl.ds(s * TPN, TPN)],
                            out_hbm.at[ch, pl.ds(s * TPN, TPN)])
            plsc.subcore_barrier()

    return _scat


def _scat4(y1c, srcr, dstr, zerc):
    return _make_scatter(NCH1)(y1c, srcr, dstr, zerc)


def _scat2(y2c, srcr, dstr, zerc):
    return _make_scatter(NCH2)(y2c, srcr, dstr, zerc)


def _dinv_from(deg_blk):
    deg = deg_blk[0, :, 0:1] + deg_blk[1, :, 0:1] + 1.0
    return lax.rsqrt(deg)


def _mm1(xp, W1):
    def body(x_ref, w_ref, o_ref):
        o_ref[...] = jnp.dot(x_ref[...], w_ref[...],
                             preferred_element_type=jnp.float32)

    return pl.pallas_call(
        body,
        grid=(NBLK,),
        in_specs=[pl.BlockSpec((BN, 128), lambda i: (i, 0)),
                  pl.BlockSpec((128, 128), lambda i: (0, 0))],
        out_specs=pl.BlockSpec((BN, 128), lambda i: (i, 0)),
        out_shape=jax.ShapeDtypeStruct((NPAD, 128), jnp.float32),
    )(xp, W1)


def _ychunk(xw, degp):
    def body(x_ref, d_ref, o_ref):
        y = _dinv_from(d_ref[...]) * x_ref[...]
        for c in range(NCH1):
            o_ref[c] = y[:, CW * c:CW * (c + 1)]

    return pl.pallas_call(
        body,
        grid=(NBLK,),
        in_specs=[pl.BlockSpec((BN, 128), lambda i: (i, 0)),
                  pl.BlockSpec((2, BN, DEGW), lambda i: (0, i, 0))],
        out_specs=pl.BlockSpec((NCH1, BN, CW), lambda i: (0, i, 0)),
        out_shape=jax.ShapeDtypeStruct((NCH1, NPAD, CW), jnp.float32),
    )(xw, degp)


def _stat(sseg, yc, degp, br, nch):
    def body(s_ref, y_ref, d_ref, b_ref, h_ref, sum_ref):
        i = pl.program_id(0)
        dinv = _dinv_from(d_ref[...])
        bb = b_ref[...]
        rows = lax.broadcasted_iota(jnp.int32, (BN, 1), 0) + i * BN
        mask = rows < N
        parts = []
        for c in range(nch):
            h = dinv * (s_ref[c] + y_ref[c]) + bb[c:c + 1]
            e = jnp.where(h > 0, h, jnp.exp(h) - 1.0)
            h_ref[c] = e
            em = jnp.where(mask, e, 0.0)
            parts.append(jnp.concatenate(
                [jnp.sum(em, axis=0, keepdims=True),
                 jnp.sum(em * em, axis=0, keepdims=True)], axis=0)[None])

        @pl.when(i == 0)
        def _():
            sum_ref[...] = jnp.zeros_like(sum_ref)

        sum_ref[...] += jnp.concatenate(parts, axis=0)

    return pl.pallas_call(
        body,
        grid=(NBLK,),
        in_specs=[pl.BlockSpec((nch, BN, CW), lambda i: (0, i, 0)),
                  pl.BlockSpec((nch, BN, CW), lambda i: (0, i, 0)),
                  pl.BlockSpec((2, BN, DEGW), lambda i: (0, i, 0)),
                  pl.BlockSpec((nch, CW), lambda i: (0, 0))],
        out_specs=[pl.BlockSpec((nch, BN, CW), lambda i: (0, i, 0)),
                   pl.BlockSpec((nch, 2, CW), lambda i: (0, 0, 0))],
        out_shape=[jax.ShapeDtypeStruct((nch, NPAD, CW), jnp.float32),
                   jax.ShapeDtypeStruct((nch, 2, CW), jnp.float32)],
    )(sseg, yc, degp, br)


def _gnorm_cols(hc, m, ms, w, b, eh2):
    var = eh2 - m * m * ms * (2.0 - ms)
    return (hc - m * ms) * (w * lax.rsqrt(var + 1e-5)) + b


def _mm2(h1c, sums, degp, gw, gb, gms, W2r):
    def body(h_ref, sm_ref, d_ref, gw_ref, gb_ref, gms_ref, w_ref, o_ref):
        dinv = _dinv_from(d_ref[...])
        h = h_ref[...]
        sm = sm_ref[...]
        g_w = gw_ref[...]
        g_b = gb_ref[...]
        g_ms = gms_ref[...]
        w = w_ref[...]
        acc = jnp.zeros((BN, 64), jnp.float32)
        for c in range(NCH1):
            m = sm[c, 0:1, :] * (1.0 / N)
            eh2 = sm[c, 1:2, :] * (1.0 / N)
            gc = _gnorm_cols(h[c], m, g_ms[c:c + 1], g_w[c:c + 1],
                             g_b[c:c + 1], eh2)
            acc = acc + jnp.dot(gc, w[c], preferred_element_type=jnp.float32)
        y2 = dinv * acc
        for c in range(NCH2):
            o_ref[c] = y2[:, CW * c:CW * (c + 1)]

    return pl.pallas_call(
        body,
        grid=(NBLK,),
        in_specs=[pl.BlockSpec((NCH1, BN, CW), lambda i: (0, i, 0)),
                  pl.BlockSpec((NCH1, 2, CW), lambda i: (0, 0, 0)),
                  pl.BlockSpec((2, BN, DEGW), lambda i: (0, i, 0)),
                  pl.BlockSpec((NCH1, CW), lambda i: (0, 0)),
                  pl.BlockSpec((NCH1, CW), lambda i: (0, 0)),
                  pl.BlockSpec((NCH1, CW), lambda i: (0, 0)),
                  pl.BlockSpec((NCH1, CW, 64), lambda i: (0, 0, 0))],
        out_specs=pl.BlockSpec((NCH2, BN, CW), lambda i: (0, i, 0)),
        out_shape=jax.ShapeDtypeStruct((NCH2, NPAD, CW), jnp.float32),
    )(h1c, sums, degp, gw, gb, gms, W2r)


def _norm(h2c, sums, gw, gb, gms, nch):
    def body(h_ref, sm_ref, gw_ref, gb_ref, gms_ref, o_ref):
        sm = sm_ref[...]
        g_w = gw_ref[...]
        g_b = gb_ref[...]
        g_ms = gms_ref[...]
        for c in range(nch):
            m = sm[c, 0:1, :] * (1.0 / N)
            eh2 = sm[c, 1:2, :] * (1.0 / N)
            o_ref[c] = _gnorm_cols(h_ref[c], m, g_ms[c:c + 1], g_w[c:c + 1],
                                   g_b[c:c + 1], eh2)

    return pl.pallas_call(
        body,
        grid=(NBLK,),
        in_specs=[pl.BlockSpec((nch, BN, CW), lambda i: (0, i, 0)),
                  pl.BlockSpec((nch, 2, CW), lambda i: (0, 0, 0)),
                  pl.BlockSpec((nch, CW), lambda i: (0, 0)),
                  pl.BlockSpec((nch, CW), lambda i: (0, 0)),
                  pl.BlockSpec((nch, CW), lambda i: (0, 0))],
        out_specs=pl.BlockSpec((nch, BN, CW), lambda i: (0, i, 0)),
        out_shape=jax.ShapeDtypeStruct((nch, NPAD, CW), jnp.float32),
    )(h2c, sums, gw, gb, gms)


def _head(A, Wstack, b1r, bng, bnb, bnm, bnv, W2h, b2r):
    KA = 228 * CW

    def body(a_ref, w_ref, b1_ref, g_ref, bb_ref, m_ref, v_ref,
             w2_ref, b2_ref, o_ref, acc_ref):
        c = pl.program_id(0)

        @pl.when(c == 0)
        def _():
            acc_ref[...] = jnp.zeros_like(acc_ref)

        acc_ref[...] += jnp.dot(a_ref[0], w_ref[0],
                                preferred_element_type=jnp.float32)

        @pl.when(c == NCH2 - 1)
        def _():
            z = acc_ref[...] + b1_ref[...]
            z = jnp.where(z > 0, z, jnp.exp(z) - 1.0)
            z = (z - m_ref[...]) * (g_ref[...] * lax.rsqrt(v_ref[...] + 1e-5)) \
                + bb_ref[...]
            o_ref[...] = jnp.dot(z, w2_ref[...],
                                 preferred_element_type=jnp.float32) \
                + b2_ref[...]

    return pl.pallas_call(
        body,
        grid=(NCH2,),
        in_specs=[pl.BlockSpec((1, 150, KA), lambda c: (c, 0, 0)),
                  pl.BlockSpec((1, KA, 128), lambda c: (c, 0, 0)),
                  pl.BlockSpec((1, 128), lambda c: (0, 0)),
                  pl.BlockSpec((1, 128), lambda c: (0, 0)),
                  pl.BlockSpec((1, 128), lambda c: (0, 0)),
                  pl.BlockSpec((1, 128), lambda c: (0, 0)),
                  pl.BlockSpec((1, 128), lambda c: (0, 0)),
                  pl.BlockSpec((128, 10), lambda c: (0, 0)),
                  pl.BlockSpec((1, 10), lambda c: (0, 0))],
        out_specs=pl.BlockSpec((150, 10), lambda c: (0, 0)),
        out_shape=jax.ShapeDtypeStruct((150, 10), jnp.float32),
        scratch_shapes=[pltpu.VMEM((150, 128), jnp.float32)],
    )(A, Wstack, b1r, bng, bnb, bnm, bnv, W2h, b2r)


def kernel(x, edge_index, W1, b1, W2, b2, gn1_w, gn1_b, gn1_ms,
           gn2_w, gn2_b, gn2_ms, lin1_W, lin1_b, bn_g, bn_b, bn_m, bn_v,
           lin2_W, lin2_b):
    src = edge_index[0].astype(jnp.int32)
    dst = edge_index[1].astype(jnp.int32)
    pad_e = EPAD - E
    pad_dst = N + jnp.arange(pad_e, dtype=jnp.int32) % (NPAD - N)
    srcr = jnp.concatenate([src, jnp.zeros((pad_e,), jnp.int32)])
    dstr = jnp.concatenate([dst, pad_dst])
    xp = jnp.pad(x, ((0, NPAD - N), (0, 0)))
    ones16 = jnp.ones((128, DEGW), jnp.float32)
    zer16 = jnp.zeros((TPN, DEGW), jnp.float32)
    zerc = jnp.zeros((TPN, CW), jnp.float32)

    srcr2 = srcr.reshape(NROWS, 128)
    dstr2 = dstr.reshape(NROWS, 128)

    degp = _deg_sc(dstr, ones16, zer16)
    xw = _mm1(xp, W1)
    y1c = _ychunk(xw, degp)
    s1 = _scat4(y1c, srcr2, dstr2, zerc)
    h1c, sums1 = _stat(s1, y1c, degp, b1.reshape(NCH1, CW), NCH1)
    y2c = _mm2(h1c, sums1, degp, gn1_w.reshape(NCH1, CW),
               gn1_b.reshape(NCH1, CW), gn1_ms.reshape(NCH1, CW),
               W2.reshape(NCH1, CW, 64))
    s2 = _scat2(y2c, srcr2, dstr2, zerc)
    h2c, sums2 = _stat(s2, y2c, degp, b2.reshape(NCH2, CW), NCH2)
    g2c = _norm(h2c, sums2, gn2_w.reshape(NCH2, CW), gn2_b.reshape(NCH2, CW),
                gn2_ms.reshape(NCH2, CW), NCH2)
    A = g2c[:, :N, :].reshape(NCH2, 150, 228 * CW)
    Wstack = lin1_W.reshape(228, NCH2, CW, 128).transpose(1, 0, 2, 3) \
        .reshape(NCH2, 228 * CW, 128)
    return _head(A, Wstack, lin1_b.reshape(1, 128), bn_g.reshape(1, 128),
                 bn_b.reshape(1, 128), bn_m.reshape(1, 128),
                 bn_v.reshape(1, 128), lin2_W, lin2_b.reshape(1, 10))


# speedup vs baseline: 1.9901x; 1.0917x over previous
import functools

import jax
import jax.numpy as jnp
from jax import lax
from jax.experimental import pallas as pl
from jax.experimental.pallas import tpu as pltpu
from jax.experimental.pallas import tpu_sc as plsc

N = 34200
E = 547200
NPAD = 34304
EPAD = 548864
NROWS = EPAD // 128
TPN = NPAD // 16
CW = 16
NCH1 = 128 // CW
NCH2 = 64 // CW
DEGW = 16
BN = 256
NBLK = NPAD // BN


def _sc_mesh():
    return plsc.VectorSubcoreMesh(core_axis_name="c", subcore_axis_name="s",
                                  num_cores=2, num_subcores=16)


@functools.cache
def _make_deg():
    @functools.partial(
        pl.kernel,
        out_type=jax.ShapeDtypeStruct((2, NPAD, DEGW), jnp.float32),
        mesh=_sc_mesh(),
        compiler_params=pltpu.CompilerParams(use_tc_tiling_on_sc=False),
        scratch_types=[
            pltpu.VMEM_SHARED((NPAD, DEGW), jnp.float32),
            pltpu.VMEM((TPN, DEGW), jnp.float32),
            pltpu.VMEM((128, DEGW), jnp.float32),
            pltpu.VMEM((128,), jnp.int32),
        ],
    )
    def _deg(dst_hbm, ones_hbm, zer_hbm, out_hbm, acc, zbuf, ones_v, dst_v):
        c = lax.axis_index("c")
        s = lax.axis_index("s")
        pltpu.sync_copy(zer_hbm, zbuf)
        pltpu.sync_copy(ones_hbm, ones_v)
        pltpu.sync_copy(zbuf, acc.at[pl.ds(s * TPN, TPN)])
        plsc.subcore_barrier()
        rows_per_tile = NROWS // 2 // 16
        base = (c * (NROWS // 2) + s * rows_per_tile) * 128

        def body(r, carry):
            pltpu.sync_copy(dst_hbm.at[pl.ds(base + r * 128, 128)], dst_v)
            pltpu.sync_copy(ones_v, acc.at[dst_v], add=True)
            return carry

        lax.fori_loop(0, rows_per_tile, body, 0)
        plsc.subcore_barrier()
        pltpu.sync_copy(acc.at[pl.ds(s * TPN, TPN)],
                        out_hbm.at[c, pl.ds(s * TPN, TPN)])

    return _deg


def _deg_sc(dstr, ones16, zer16):
    return _make_deg()(dstr, ones16, zer16)


RB = 4
RPT = NROWS // 16
NB = RPT // RB


@functools.cache
def _make_scatter(n_chunks):
    cpc = n_chunks // 2
    FW = n_chunks * CW

    @functools.partial(
        pl.kernel,
        out_type=jax.ShapeDtypeStruct((n_chunks, NPAD, CW), jnp.float32),
        mesh=_sc_mesh(),
        compiler_params=pltpu.CompilerParams(use_tc_tiling_on_sc=False),
        scratch_types=[
            pltpu.VMEM_SHARED((NPAD, CW), jnp.float32),
            pltpu.VMEM((TPN, CW), jnp.float32),
            pltpu.VMEM((2, RB, 128), jnp.int32),
            pltpu.VMEM((2, RB, 128), jnp.int32),
            pltpu.VMEM((2, RB, 128, CW), jnp.float32),
            pltpu.SemaphoreType.DMA,
            pltpu.SemaphoreType.DMA,
        ],
    )
    def _scat(y_hbm, src2d_hbm, dst2d_hbm, zer_hbm, out_hbm,
              acc, zbuf, src_v, dst_v, vals, g0, g1):
        c = lax.axis_index("c")
        s = lax.axis_index("s")
        gsem = (g0, g1)
        pltpu.sync_copy(zer_hbm, zbuf)
        base = s * RPT

        for p in range(cpc):
            ch = c * cpc + p
            tbl = y_hbm.at[ch]
            pltpu.sync_copy(zbuf, acc.at[pl.ds(s * TPN, TPN)])
            plsc.subcore_barrier()

            def load_idx(t, slot):
                pltpu.sync_copy(src2d_hbm.at[pl.ds(base + t * RB, RB)],
                                src_v.at[slot])
                pltpu.sync_copy(dst2d_hbm.at[pl.ds(base + t * RB, RB)],
                                dst_v.at[slot])

            def fire_g(slot):
                for j in range(RB):
                    pltpu.async_copy(tbl.at[src_v.at[slot, j]],
                                     vals.at[slot, j], gsem[slot])

            def wait_g(slot):
                for j in range(RB):
                    pltpu.make_async_copy(tbl.at[src_v.at[slot, j]],
                                          vals.at[slot, j], gsem[slot]).wait()

            def scat_sync(slot):
                for j in range(RB):
                    pltpu.sync_copy(vals.at[slot, j],
                                    acc.at[dst_v.at[slot, j]], add=True)

            def block(t, slot, prefetch):
                if prefetch:
                    load_idx(t + 1, 1 - slot)
                    fire_g(1 - slot)
                wait_g(slot)
                scat_sync(slot)

            load_idx(0, 0)
            fire_g(0)

            def body(t2, carry):
                t = 2 * t2
                block(t, 0, True)
                block(t + 1, 1, True)
                return carry

            lax.fori_loop(0, (NB - 1) // 2, body, 0)
            block(NB - 1, 0, False)

            plsc.subcore_barrier()
            pltpu.sync_copy(acc.at[pl.ds(s * TPN, TPN)],
                            out_hbm.at[ch, pl.ds(s * TPN, TPN)])
            plsc.subcore_barrier()

    return _scat


def _scat4(y1c, srcr, dstr, zerc):
    return _make_scatter(NCH1)(y1c, srcr, dstr, zerc)


def _scat2(y2c, srcr, dstr, zerc):
    return _make_scatter(NCH2)(y2c, srcr, dstr, zerc)


def _dinv_from(deg_blk):
    deg = deg_blk[0, :, 0:1] + deg_blk[1, :, 0:1] + 1.0
    return lax.rsqrt(deg)


def _mm1(xp, W1):
    def body(x_ref, w_ref, o_ref):
        o_ref[...] = jnp.dot(x_ref[...], w_ref[...],
                             preferred_element_type=jnp.float32)

    return pl.pallas_call(
        body,
        grid=(NBLK,),
        in_specs=[pl.BlockSpec((BN, 128), lambda i: (i, 0)),
                  pl.BlockSpec((128, 128), lambda i: (0, 0))],
        out_specs=pl.BlockSpec((BN, 128), lambda i: (i, 0)),
        out_shape=jax.ShapeDtypeStruct((NPAD, 128), jnp.float32),
    )(xp, W1)


def _ychunk(xw, degp):
    def body(x_ref, d_ref, o_ref):
        y = _dinv_from(d_ref[...]) * x_ref[...]
        for c in range(NCH1):
            o_ref[c] = y[:, CW * c:CW * (c + 1)]

    return pl.pallas_call(
        body,
        grid=(NBLK,),
        in_specs=[pl.BlockSpec((BN, 128), lambda i: (i, 0)),
                  pl.BlockSpec((2, BN, DEGW), lambda i: (0, i, 0))],
        out_specs=pl.BlockSpec((NCH1, BN, CW), lambda i: (0, i, 0)),
        out_shape=jax.ShapeDtypeStruct((NCH1, NPAD, CW), jnp.float32),
    )(xw, degp)


def _stat(sseg, yc, degp, br, nch):
    FW = nch * CW

    def body(s_ref, y_ref, d_ref, b_ref, h_ref, sum_ref):
        i = pl.program_id(0)
        dinv = _dinv_from(d_ref[...])
        ycat = jnp.concatenate([y_ref[c] for c in range(nch)], axis=1)
        h = dinv * (s_ref[...] + ycat) + b_ref[...]
        e = jnp.where(h > 0, h, jnp.exp(h) - 1.0)
        h_ref[...] = e
        rows = lax.broadcasted_iota(jnp.int32, (BN, 1), 0) + i * BN
        em = jnp.where(rows < N, e, 0.0)
        both = jnp.concatenate(
            [jnp.sum(em, axis=0, keepdims=True),
             jnp.sum(em * em, axis=0, keepdims=True)], axis=0)

        @pl.when(i == 0)
        def _():
            sum_ref[...] = jnp.zeros_like(sum_ref)

        sum_ref[...] += both

    return pl.pallas_call(
        body,
        grid=(NBLK,),
        in_specs=[pl.BlockSpec((BN, FW), lambda i: (i, 0)),
                  pl.BlockSpec((nch, BN, CW), lambda i: (0, i, 0)),
                  pl.BlockSpec((2, BN, DEGW), lambda i: (0, i, 0)),
                  pl.BlockSpec((1, FW), lambda i: (0, 0))],
        out_specs=[pl.BlockSpec((BN, FW), lambda i: (i, 0)),
                   pl.BlockSpec((2, FW), lambda i: (0, 0))],
        out_shape=[jax.ShapeDtypeStruct((NPAD, FW), jnp.float32),
                   jax.ShapeDtypeStruct((2, FW), jnp.float32)],
    )(sseg, yc, degp, br)


def _gnorm_cols(hc, m, ms, w, b, eh2):
    var = eh2 - m * m * ms * (2.0 - ms)
    return (hc - m * ms) * (w * lax.rsqrt(var + 1e-5)) + b


def _mm2(h1, sums, degp, gw, gb, gms, W2):
    def body(h_ref, sm_ref, d_ref, gw_ref, gb_ref, gms_ref, w_ref, o_ref):
        dinv = _dinv_from(d_ref[...])
        sm = sm_ref[...]
        m = sm[0:1] * (1.0 / N)
        eh2 = sm[1:2] * (1.0 / N)
        g = _gnorm_cols(h_ref[...], m, gms_ref[...], gw_ref[...],
                        gb_ref[...], eh2)
        y2 = dinv * jnp.dot(g, w_ref[...], preferred_element_type=jnp.float32)
        for c in range(NCH2):
            o_ref[c] = y2[:, CW * c:CW * (c + 1)]

    return pl.pallas_call(
        body,
        grid=(NBLK,),
        in_specs=[pl.BlockSpec((BN, 128), lambda i: (i, 0)),
                  pl.BlockSpec((2, 128), lambda i: (0, 0)),
                  pl.BlockSpec((2, BN, DEGW), lambda i: (0, i, 0)),
                  pl.BlockSpec((1, 128), lambda i: (0, 0)),
                  pl.BlockSpec((1, 128), lambda i: (0, 0)),
                  pl.BlockSpec((1, 128), lambda i: (0, 0)),
                  pl.BlockSpec((128, 64), lambda i: (0, 0))],
        out_specs=pl.BlockSpec((NCH2, BN, CW), lambda i: (0, i, 0)),
        out_shape=jax.ShapeDtypeStruct((NCH2, NPAD, CW), jnp.float32),
    )(h1, sums, degp, gw, gb, gms, W2)


def _norm(h2, sums, gw, gb, gms, fw):
    def body(h_ref, sm_ref, gw_ref, gb_ref, gms_ref, o_ref):
        sm = sm_ref[...]
        m = sm[0:1] * (1.0 / N)
        eh2 = sm[1:2] * (1.0 / N)
        o_ref[...] = _gnorm_cols(h_ref[...], m, gms_ref[...], gw_ref[...],
                                 gb_ref[...], eh2)

    return pl.pallas_call(
        body,
        grid=(NBLK,),
        in_specs=[pl.BlockSpec((BN, fw), lambda i: (i, 0)),
                  pl.BlockSpec((2, fw), lambda i: (0, 0)),
                  pl.BlockSpec((1, fw), lambda i: (0, 0)),
                  pl.BlockSpec((1, fw), lambda i: (0, 0)),
                  pl.BlockSpec((1, fw), lambda i: (0, 0))],
        out_specs=pl.BlockSpec((BN, fw), lambda i: (i, 0)),
        out_shape=jax.ShapeDtypeStruct((NPAD, fw), jnp.float32),
    )(h2, sums, gw, gb, gms)


def _head(A, W1h, b1r, bng, bnb, bnm, bnv, W2h, b2r):
    KB = 2432
    KT = 14592 // KB

    def body(a_ref, w_ref, b1_ref, g_ref, bb_ref, m_ref, v_ref,
             w2_ref, b2_ref, o_ref, acc_ref):
        k = pl.program_id(0)

        @pl.when(k == 0)
        def _():
            acc_ref[...] = jnp.zeros_like(acc_ref)

        acc_ref[...] += jnp.dot(a_ref[...], w_ref[...],
                                preferred_element_type=jnp.float32)

        @pl.when(k == KT - 1)
        def _():
            z = acc_ref[...] + b1_ref[...]
            z = jnp.where(z > 0, z, jnp.exp(z) - 1.0)
            z = (z - m_ref[...]) * (g_ref[...] * lax.rsqrt(v_ref[...] + 1e-5)) \
                + bb_ref[...]
            o_ref[...] = jnp.dot(z, w2_ref[...],
                                 preferred_element_type=jnp.float32) \
                + b2_ref[...]

    return pl.pallas_call(
        body,
        grid=(KT,),
        in_specs=[pl.BlockSpec((150, KB), lambda k: (0, k)),
                  pl.BlockSpec((KB, 128), lambda k: (k, 0)),
                  pl.BlockSpec((1, 128), lambda k: (0, 0)),
                  pl.BlockSpec((1, 128), lambda k: (0, 0)),
                  pl.BlockSpec((1, 128), lambda k: (0, 0)),
                  pl.BlockSpec((1, 128), lambda k: (0, 0)),
                  pl.BlockSpec((1, 128), lambda k: (0, 0)),
                  pl.BlockSpec((128, 10), lambda k: (0, 0)),
                  pl.BlockSpec((1, 10), lambda k: (0, 0))],
        out_specs=pl.BlockSpec((150, 10), lambda k: (0, 0)),
        out_shape=jax.ShapeDtypeStruct((150, 10), jnp.float32),
        scratch_shapes=[pltpu.VMEM((150, 128), jnp.float32)],
    )(A, W1h, b1r, bng, bnb, bnm, bnv, W2h, b2r)


def kernel(x, edge_index, W1, b1, W2, b2, gn1_w, gn1_b, gn1_ms,
           gn2_w, gn2_b, gn2_ms, lin1_W, lin1_b, bn_g, bn_b, bn_m, bn_v,
           lin2_W, lin2_b):
    src = edge_index[0].astype(jnp.int32)
    dst = edge_index[1].astype(jnp.int32)
    pad_e = EPAD - E
    pad_dst = N + jnp.arange(pad_e, dtype=jnp.int32) % (NPAD - N)
    srcr = jnp.concatenate([src, jnp.zeros((pad_e,), jnp.int32)])
    dstr = jnp.concatenate([dst, pad_dst])
    xp = jnp.pad(x, ((0, NPAD - N), (0, 0)))
    ones16 = jnp.ones((128, DEGW), jnp.float32)
    zer16 = jnp.zeros((TPN, DEGW), jnp.float32)
    zerc = jnp.zeros((TPN, CW), jnp.float32)

    srcr2 = srcr.reshape(NROWS, 128)
    dstr2 = dstr.reshape(NROWS, 128)

    degp = _deg_sc(dstr, ones16, zer16)
    xw = _mm1(xp, W1)
    y1c = _ychunk(xw, degp)
    s1 = _scat4(y1c, srcr2, dstr2, zerc)
    s1n = jnp.moveaxis(s1, 0, 1).reshape(NPAD, 128)
    h1, sums1 = _stat(s1n, y1c, degp, b1.reshape(1, 128), NCH1)
    y2c = _mm2(h1, sums1, degp, gn1_w.reshape(1, 128),
               gn1_b.reshape(1, 128), gn1_ms.reshape(1, 128), W2)
    s2 = _scat2(y2c, srcr2, dstr2, zerc)
    s2n = jnp.moveaxis(s2, 0, 1).reshape(NPAD, 64)
    h2, sums2 = _stat(s2n, y2c, degp, b2.reshape(1, 64), NCH2)
    g2 = _norm(h2, sums2, gn2_w.reshape(1, 64), gn2_b.reshape(1, 64),
               gn2_ms.reshape(1, 64), 64)
    A = g2[:N].reshape(150, 228 * 64)
    return _head(A, lin1_W, lin1_b.reshape(1, 128), bn_g.reshape(1, 128),
                 bn_b.reshape(1, 128), bn_m.reshape(1, 128),
                 bn_v.reshape(1, 128), lin2_W, lin2_b.reshape(1, 10))


# speedup vs baseline: 2.1288x; 1.0697x over previous
import functools

import jax
import jax.numpy as jnp
from jax import lax
from jax.experimental import pallas as pl
from jax.experimental.pallas import tpu as pltpu
from jax.experimental.pallas import tpu_sc as plsc

N = 34200
E = 547200
NPAD = 34304
EPAD = 548864
NROWS = EPAD // 128
TPN = NPAD // 16
CW = 16
NCH1 = 128 // CW
NCH2 = 64 // CW
DEGW = 16
BN = 256
NBLK = NPAD // BN


def _sc_mesh():
    return plsc.VectorSubcoreMesh(core_axis_name="c", subcore_axis_name="s",
                                  num_cores=2, num_subcores=16)


@functools.cache
def _make_deg():
    @functools.partial(
        pl.kernel,
        out_type=jax.ShapeDtypeStruct((2, NPAD, DEGW), jnp.float32),
        mesh=_sc_mesh(),
        compiler_params=pltpu.CompilerParams(use_tc_tiling_on_sc=False),
        scratch_types=[
            pltpu.VMEM_SHARED((NPAD, DEGW), jnp.float32),
            pltpu.VMEM((TPN, DEGW), jnp.float32),
            pltpu.VMEM((128, DEGW), jnp.float32),
            pltpu.VMEM((128,), jnp.int32),
        ],
    )
    def _deg(dst_hbm, ones_hbm, zer_hbm, out_hbm, acc, zbuf, ones_v, dst_v):
        c = lax.axis_index("c")
        s = lax.axis_index("s")
        pltpu.sync_copy(zer_hbm, zbuf)
        pltpu.sync_copy(ones_hbm, ones_v)
        pltpu.sync_copy(zbuf, acc.at[pl.ds(s * TPN, TPN)])
        plsc.subcore_barrier()
        rows_per_tile = NROWS // 2 // 16
        base = (c * (NROWS // 2) + s * rows_per_tile) * 128

        def body(r, carry):
            pltpu.sync_copy(dst_hbm.at[pl.ds(base + r * 128, 128)], dst_v)
            pltpu.sync_copy(ones_v, acc.at[dst_v], add=True)
            return carry

        lax.fori_loop(0, rows_per_tile, body, 0)
        plsc.subcore_barrier()
        pltpu.sync_copy(acc.at[pl.ds(s * TPN, TPN)],
                        out_hbm.at[c, pl.ds(s * TPN, TPN)])

    return _deg


def _deg_sc(dstr, ones16, zer16):
    return _make_deg()(dstr, ones16, zer16)


RB = 4
RPT = NROWS // 16
NB = RPT // RB


@functools.cache
def _make_scatter(n_chunks):
    cpc = n_chunks // 2
    FW = n_chunks * CW

    @functools.partial(
        pl.kernel,
        out_type=jax.ShapeDtypeStruct((NPAD, FW), jnp.float32),
        mesh=_sc_mesh(),
        compiler_params=pltpu.CompilerParams(use_tc_tiling_on_sc=False),
        scratch_types=[
            pltpu.VMEM_SHARED((NPAD, CW), jnp.float32),
            pltpu.VMEM((TPN, CW), jnp.float32),
            pltpu.VMEM((2, RB, 128), jnp.int32),
            pltpu.VMEM((2, RB, 128), jnp.int32),
            pltpu.VMEM((2, RB, 128, CW), jnp.float32),
            pltpu.SemaphoreType.DMA,
            pltpu.SemaphoreType.DMA,
        ],
    )
    def _scat(y_hbm, src2d_hbm, dst2d_hbm, zer_hbm, out_hbm,
              acc, zbuf, src_v, dst_v, vals, g0, g1):
        c = lax.axis_index("c")
        s = lax.axis_index("s")
        gsem = (g0, g1)
        pltpu.sync_copy(zer_hbm, zbuf)
        base = s * RPT

        for p in range(cpc):
            ch = c * cpc + p
            tbl = y_hbm.at[ch]
            pltpu.sync_copy(zbuf, acc.at[pl.ds(s * TPN, TPN)])
            plsc.subcore_barrier()

            def load_idx(t, slot):
                pltpu.sync_copy(src2d_hbm.at[pl.ds(base + t * RB, RB)],
                                src_v.at[slot])
                pltpu.sync_copy(dst2d_hbm.at[pl.ds(base + t * RB, RB)],
                                dst_v.at[slot])

            def fire_g(slot):
                for j in range(RB):
                    pltpu.async_copy(tbl.at[src_v.at[slot, j]],
                                     vals.at[slot, j], gsem[slot])

            def wait_g(slot):
                for j in range(RB):
                    pltpu.make_async_copy(tbl.at[src_v.at[slot, j]],
                                          vals.at[slot, j], gsem[slot]).wait()

            def scat_sync(slot):
                for j in range(RB):
                    pltpu.sync_copy(vals.at[slot, j],
                                    acc.at[dst_v.at[slot, j]], add=True)

            def block(t, slot, prefetch):
                if prefetch:
                    load_idx(t + 1, 1 - slot)
                    fire_g(1 - slot)
                wait_g(slot)
                scat_sync(slot)

            load_idx(0, 0)
            fire_g(0)

            def body(t2, carry):
                t = 2 * t2
                block(t, 0, True)
                block(t + 1, 1, True)
                return carry

            lax.fori_loop(0, (NB - 1) // 2, body, 0)
            block(NB - 1, 0, False)

            plsc.subcore_barrier()
            pltpu.sync_copy(acc.at[pl.ds(s * TPN, TPN)],
                            out_hbm.at[pl.ds(s * TPN, TPN),
                                       pl.ds(ch * CW, CW)])
            plsc.subcore_barrier()

    return _scat


def _scat4(y1c, srcr, dstr, zerc):
    return _make_scatter(NCH1)(y1c, srcr, dstr, zerc)


def _scat2(y2c, srcr, dstr, zerc):
    return _make_scatter(NCH2)(y2c, srcr, dstr, zerc)


def _dinv_from(deg_blk):
    deg = deg_blk[0, :, 0:1] + deg_blk[1, :, 0:1] + 1.0
    return lax.rsqrt(deg)


def _mm1(xp, W1):
    def body(x_ref, w_ref, o_ref):
        o_ref[...] = jnp.dot(x_ref[...], w_ref[...],
                             preferred_element_type=jnp.float32)

    return pl.pallas_call(
        body,
        grid=(NBLK,),
        in_specs=[pl.BlockSpec((BN, 128), lambda i: (i, 0)),
                  pl.BlockSpec((128, 128), lambda i: (0, 0))],
        out_specs=pl.BlockSpec((BN, 128), lambda i: (i, 0)),
        out_shape=jax.ShapeDtypeStruct((NPAD, 128), jnp.float32),
    )(xp, W1)


def _ychunk(xw, degp):
    def body(x_ref, d_ref, o_ref):
        y = _dinv_from(d_ref[...]) * x_ref[...]
        for c in range(NCH1):
            o_ref[c] = y[:, CW * c:CW * (c + 1)]

    return pl.pallas_call(
        body,
        grid=(NBLK,),
        in_specs=[pl.BlockSpec((BN, 128), lambda i: (i, 0)),
                  pl.BlockSpec((2, BN, DEGW), lambda i: (0, i, 0))],
        out_specs=pl.BlockSpec((NCH1, BN, CW), lambda i: (0, i, 0)),
        out_shape=jax.ShapeDtypeStruct((NCH1, NPAD, CW), jnp.float32),
    )(xw, degp)


def _stat(sseg, yc, degp, br, nch):
    FW = nch * CW

    def body(s_ref, y_ref, d_ref, b_ref, h_ref, sum_ref):
        i = pl.program_id(0)
        dinv = _dinv_from(d_ref[...])
        ycat = jnp.concatenate([y_ref[c] for c in range(nch)], axis=1)
        h = dinv * (s_ref[...] + ycat) + b_ref[...]
        e = jnp.where(h > 0, h, jnp.exp(h) - 1.0)
        h_ref[...] = e
        rows = lax.broadcasted_iota(jnp.int32, (BN, 1), 0) + i * BN
        em = jnp.where(rows < N, e, 0.0)
        both = jnp.concatenate(
            [jnp.sum(em, axis=0, keepdims=True),
             jnp.sum(em * em, axis=0, keepdims=True)], axis=0)

        @pl.when(i == 0)
        def _():
            sum_ref[...] = jnp.zeros_like(sum_ref)

        sum_ref[...] += both

    return pl.pallas_call(
        body,
        grid=(NBLK,),
        in_specs=[pl.BlockSpec((BN, FW), lambda i: (i, 0)),
                  pl.BlockSpec((nch, BN, CW), lambda i: (0, i, 0)),
                  pl.BlockSpec((2, BN, DEGW), lambda i: (0, i, 0)),
                  pl.BlockSpec((1, FW), lambda i: (0, 0))],
        out_specs=[pl.BlockSpec((BN, FW), lambda i: (i, 0)),
                   pl.BlockSpec((2, FW), lambda i: (0, 0))],
        out_shape=[jax.ShapeDtypeStruct((NPAD, FW), jnp.float32),
                   jax.ShapeDtypeStruct((2, FW), jnp.float32)],
    )(sseg, yc, degp, br)


def _gnorm_cols(hc, m, ms, w, b, eh2):
    var = eh2 - m * m * ms * (2.0 - ms)
    return (hc - m * ms) * (w * lax.rsqrt(var + 1e-5)) + b


def _mm2(h1, sums, degp, gw, gb, gms, W2):
    def body(h_ref, sm_ref, d_ref, gw_ref, gb_ref, gms_ref, w_ref, o_ref):
        dinv = _dinv_from(d_ref[...])
        sm = sm_ref[...]
        m = sm[0:1] * (1.0 / N)
        eh2 = sm[1:2] * (1.0 / N)
        g = _gnorm_cols(h_ref[...], m, gms_ref[...], gw_ref[...],
                        gb_ref[...], eh2)
        y2 = dinv * jnp.dot(g, w_ref[...], preferred_element_type=jnp.float32)
        for c in range(NCH2):
            o_ref[c] = y2[:, CW * c:CW * (c + 1)]

    return pl.pallas_call(
        body,
        grid=(NBLK,),
        in_specs=[pl.BlockSpec((BN, 128), lambda i: (i, 0)),
                  pl.BlockSpec((2, 128), lambda i: (0, 0)),
                  pl.BlockSpec((2, BN, DEGW), lambda i: (0, i, 0)),
                  pl.BlockSpec((1, 128), lambda i: (0, 0)),
                  pl.BlockSpec((1, 128), lambda i: (0, 0)),
                  pl.BlockSpec((1, 128), lambda i: (0, 0)),
                  pl.BlockSpec((128, 64), lambda i: (0, 0))],
        out_specs=pl.BlockSpec((NCH2, BN, CW), lambda i: (0, i, 0)),
        out_shape=jax.ShapeDtypeStruct((NCH2, NPAD, CW), jnp.float32),
    )(h1, sums, degp, gw, gb, gms, W2)


def _norm(h2, sums, gw, gb, gms, fw):
    def body(h_ref, sm_ref, gw_ref, gb_ref, gms_ref, o_ref):
        sm = sm_ref[...]
        m = sm[0:1] * (1.0 / N)
        eh2 = sm[1:2] * (1.0 / N)
        o_ref[...] = _gnorm_cols(h_ref[...], m, gms_ref[...], gw_ref[...],
                                 gb_ref[...], eh2)

    return pl.pallas_call(
        body,
        grid=(NBLK,),
        in_specs=[pl.BlockSpec((BN, fw), lambda i: (i, 0)),
                  pl.BlockSpec((2, fw), lambda i: (0, 0)),
                  pl.BlockSpec((1, fw), lambda i: (0, 0)),
                  pl.BlockSpec((1, fw), lambda i: (0, 0)),
                  pl.BlockSpec((1, fw), lambda i: (0, 0))],
        out_specs=pl.BlockSpec((BN, fw), lambda i: (i, 0)),
        out_shape=jax.ShapeDtypeStruct((NPAD, fw), jnp.float32),
    )(h2, sums, gw, gb, gms)


def _head(A, W1h, b1r, bng, bnb, bnm, bnv, W2h, b2r):
    KB = 2432
    KT = 14592 // KB

    def body(a_ref, w_ref, b1_ref, g_ref, bb_ref, m_ref, v_ref,
             w2_ref, b2_ref, o_ref, acc_ref):
        k = pl.program_id(0)

        @pl.when(k == 0)
        def _():
            acc_ref[...] = jnp.zeros_like(acc_ref)

        acc_ref[...] += jnp.dot(a_ref[...], w_ref[...],
                                preferred_element_type=jnp.float32)

        @pl.when(k == KT - 1)
        def _():
            z = acc_ref[...] + b1_ref[...]
            z = jnp.where(z > 0, z, jnp.exp(z) - 1.0)
            z = (z - m_ref[...]) * (g_ref[...] * lax.rsqrt(v_ref[...] + 1e-5)) \
                + bb_ref[...]
            o_ref[...] = jnp.dot(z, w2_ref[...],
                                 preferred_element_type=jnp.float32) \
                + b2_ref[...]

    return pl.pallas_call(
        body,
        grid=(KT,),
        in_specs=[pl.BlockSpec((150, KB), lambda k: (0, k)),
                  pl.BlockSpec((KB, 128), lambda k: (k, 0)),
                  pl.BlockSpec((1, 128), lambda k: (0, 0)),
                  pl.BlockSpec((1, 128), lambda k: (0, 0)),
                  pl.BlockSpec((1, 128), lambda k: (0, 0)),
                  pl.BlockSpec((1, 128), lambda k: (0, 0)),
                  pl.BlockSpec((1, 128), lambda k: (0, 0)),
                  pl.BlockSpec((128, 10), lambda k: (0, 0)),
                  pl.BlockSpec((1, 10), lambda k: (0, 0))],
        out_specs=pl.BlockSpec((150, 10), lambda k: (0, 0)),
        out_shape=jax.ShapeDtypeStruct((150, 10), jnp.float32),
        scratch_shapes=[pltpu.VMEM((150, 128), jnp.float32)],
    )(A, W1h, b1r, bng, bnb, bnm, bnv, W2h, b2r)


def kernel(x, edge_index, W1, b1, W2, b2, gn1_w, gn1_b, gn1_ms,
           gn2_w, gn2_b, gn2_ms, lin1_W, lin1_b, bn_g, bn_b, bn_m, bn_v,
           lin2_W, lin2_b):
    src = edge_index[0].astype(jnp.int32)
    dst = edge_index[1].astype(jnp.int32)
    pad_e = EPAD - E
    pad_dst = N + jnp.arange(pad_e, dtype=jnp.int32) % (NPAD - N)
    srcr = jnp.concatenate([src, jnp.zeros((pad_e,), jnp.int32)])
    dstr = jnp.concatenate([dst, pad_dst])
    xp = jnp.pad(x, ((0, NPAD - N), (0, 0)))
    ones16 = jnp.ones((128, DEGW), jnp.float32)
    zer16 = jnp.zeros((TPN, DEGW), jnp.float32)
    zerc = jnp.zeros((TPN, CW), jnp.float32)

    srcr2 = srcr.reshape(NROWS, 128)
    dstr2 = dstr.reshape(NROWS, 128)

    degp = _deg_sc(dstr, ones16, zer16)
    xw = _mm1(xp, W1)
    y1c = _ychunk(xw, degp)
    s1 = _scat4(y1c, srcr2, dstr2, zerc)
    h1, sums1 = _stat(s1, y1c, degp, b1.reshape(1, 128), NCH1)
    y2c = _mm2(h1, sums1, degp, gn1_w.reshape(1, 128),
               gn1_b.reshape(1, 128), gn1_ms.reshape(1, 128), W2)
    s2 = _scat2(y2c, srcr2, dstr2, zerc)
    h2, sums2 = _stat(s2, y2c, degp, b2.reshape(1, 64), NCH2)
    g2 = _norm(h2, sums2, gn2_w.reshape(1, 64), gn2_b.reshape(1, 64),
               gn2_ms.reshape(1, 64), 64)
    A = g2[:N].reshape(150, 228 * 64)
    return _head(A, lin1_W, lin1_b.reshape(1, 128), bn_g.reshape(1, 128),
                 bn_b.reshape(1, 128), bn_m.reshape(1, 128),
                 bn_v.reshape(1, 128), lin2_W, lin2_b.reshape(1, 10))


# speedup vs baseline: 2.4355x; 1.1441x over previous
import functools

import jax
import jax.numpy as jnp
from jax import lax
from jax.experimental import pallas as pl
from jax.experimental.pallas import tpu as pltpu
from jax.experimental.pallas import tpu_sc as plsc

N = 34200
E = 547200
NPAD = 34304
EPAD = 548864
NROWS = EPAD // 128
TPN = NPAD // 16
CW = 16
NCH1 = 128 // CW
NCH2 = 64 // CW
DEGW = 16
BN = 256
NBLK = NPAD // BN


def _sc_mesh():
    return plsc.VectorSubcoreMesh(core_axis_name="c", subcore_axis_name="s",
                                  num_cores=2, num_subcores=16)


@functools.cache
def _make_deg():
    @functools.partial(
        pl.kernel,
        out_type=jax.ShapeDtypeStruct((2, NPAD, DEGW), jnp.float32),
        mesh=_sc_mesh(),
        compiler_params=pltpu.CompilerParams(use_tc_tiling_on_sc=False),
        scratch_types=[
            pltpu.VMEM_SHARED((NPAD, DEGW), jnp.float32),
            pltpu.VMEM((TPN, DEGW), jnp.float32),
            pltpu.VMEM((128, DEGW), jnp.float32),
            pltpu.VMEM((128,), jnp.int32),
        ],
    )
    def _deg(dst_hbm, ones_hbm, zer_hbm, out_hbm, acc, zbuf, ones_v, dst_v):
        c = lax.axis_index("c")
        s = lax.axis_index("s")
        pltpu.sync_copy(zer_hbm, zbuf)
        pltpu.sync_copy(ones_hbm, ones_v)
        pltpu.sync_copy(zbuf, acc.at[pl.ds(s * TPN, TPN)])
        plsc.subcore_barrier()
        rows_per_tile = NROWS // 2 // 16
        base = (c * (NROWS // 2) + s * rows_per_tile) * 128

        def body(r, carry):
            pltpu.sync_copy(dst_hbm.at[pl.ds(base + r * 128, 128)], dst_v)
            pltpu.sync_copy(ones_v, acc.at[dst_v], add=True)
            return carry

        lax.fori_loop(0, rows_per_tile, body, 0)
        plsc.subcore_barrier()
        pltpu.sync_copy(acc.at[pl.ds(s * TPN, TPN)],
                        out_hbm.at[c, pl.ds(s * TPN, TPN)])

    return _deg


def _deg_sc(dstr, ones16, zer16):
    return _make_deg()(dstr, ones16, zer16)


RB = 4
RPT = NROWS // 16
NB = RPT // RB


@functools.cache
def _make_scatter(n_chunks):
    cpc = n_chunks // 2
    FW = n_chunks * CW

    @functools.partial(
        pl.kernel,
        out_type=jax.ShapeDtypeStruct((NPAD, FW), jnp.float32),
        mesh=_sc_mesh(),
        compiler_params=pltpu.CompilerParams(use_tc_tiling_on_sc=False),
        scratch_types=[
            pltpu.VMEM_SHARED((NPAD, CW), jnp.float32),
            pltpu.VMEM((TPN, CW), jnp.float32),
            pltpu.VMEM((2, RB, 128), jnp.int32),
            pltpu.VMEM((2, RB, 128), jnp.int32),
            pltpu.VMEM((2, RB, 128, CW), jnp.float32),
            pltpu.SemaphoreType.DMA,
            pltpu.SemaphoreType.DMA,
        ],
    )
    def _scat(y_hbm, src2d_hbm, dst2d_hbm, zer_hbm, out_hbm,
              acc, zbuf, src_v, dst_v, vals, g0, g1):
        c = lax.axis_index("c")
        s = lax.axis_index("s")
        gsem = (g0, g1)
        pltpu.sync_copy(zer_hbm, zbuf)
        base = s * RPT

        for p in range(cpc):
            ch = c * cpc + p
            tbl = y_hbm
            pltpu.sync_copy(zbuf, acc.at[pl.ds(s * TPN, TPN)])
            plsc.subcore_barrier()

            def load_idx(t, slot):
                pltpu.sync_copy(src2d_hbm.at[pl.ds(base + t * RB, RB)],
                                src_v.at[slot])
                pltpu.sync_copy(dst2d_hbm.at[pl.ds(base + t * RB, RB)],
                                dst_v.at[slot])
                for j in range(RB):
                    for k in range(128 // 16):
                        v = src_v[slot, j, pl.ds(k * 16, 16)]
                        src_v[slot, j, pl.ds(k * 16, 16)] = \
                            v * n_chunks + ch

            def fire_g(slot):
                for j in range(RB):
                    pltpu.async_copy(tbl.at[src_v.at[slot, j]],
                                     vals.at[slot, j], gsem[slot])

            def wait_g(slot):
                for j in range(RB):
                    pltpu.make_async_copy(tbl.at[src_v.at[slot, j]],
                                          vals.at[slot, j], gsem[slot]).wait()

            def scat_sync(slot):
                for j in range(RB):
                    pltpu.sync_copy(vals.at[slot, j],
                                    acc.at[dst_v.at[slot, j]], add=True)

            def block(t, slot, prefetch):
                if prefetch:
                    load_idx(t + 1, 1 - slot)
                    fire_g(1 - slot)
                wait_g(slot)
                scat_sync(slot)

            load_idx(0, 0)
            fire_g(0)

            def body(t2, carry):
                t = 2 * t2
                block(t, 0, True)
                block(t + 1, 1, True)
                return carry

            lax.fori_loop(0, (NB - 1) // 2, body, 0)
            block(NB - 1, 0, False)

            plsc.subcore_barrier()
            pltpu.sync_copy(acc.at[pl.ds(s * TPN, TPN)],
                            out_hbm.at[pl.ds(s * TPN, TPN),
                                       pl.ds(ch * CW, CW)])
            plsc.subcore_barrier()

    return _scat


def _scat4(y1c, srcr, dstr, zerc):
    return _make_scatter(NCH1)(y1c, srcr, dstr, zerc)


def _scat2(y2c, srcr, dstr, zerc):
    return _make_scatter(NCH2)(y2c, srcr, dstr, zerc)


def _dinv_from(deg_blk):
    deg = deg_blk[0, :, 0:1] + deg_blk[1, :, 0:1] + 1.0
    return lax.rsqrt(deg)


def _mm1y(xp, W1, degp):
    def body(x_ref, w_ref, d_ref, o_ref):
        o_ref[...] = _dinv_from(d_ref[...]) * jnp.dot(
            x_ref[...], w_ref[...], preferred_element_type=jnp.float32)

    return pl.pallas_call(
        body,
        grid=(NBLK,),
        in_specs=[pl.BlockSpec((BN, 128), lambda i: (i, 0)),
                  pl.BlockSpec((128, 128), lambda i: (0, 0)),
                  pl.BlockSpec((2, BN, DEGW), lambda i: (0, i, 0))],
        out_specs=pl.BlockSpec((BN, 128), lambda i: (i, 0)),
        out_shape=jax.ShapeDtypeStruct((NPAD, 128), jnp.float32),
    )(xp, W1, degp)


def _stat(sseg, yc, degp, br, nch):
    FW = nch * CW

    def body(s_ref, y_ref, d_ref, b_ref, h_ref, sum_ref):
        i = pl.program_id(0)
        dinv = _dinv_from(d_ref[...])
        h = dinv * (s_ref[...] + y_ref[...]) + b_ref[...]
        e = jnp.where(h > 0, h, jnp.exp(h) - 1.0)
        h_ref[...] = e
        rows = lax.broadcasted_iota(jnp.int32, (BN, 1), 0) + i * BN
        em = jnp.where(rows < N, e, 0.0)
        both = jnp.concatenate(
            [jnp.sum(em, axis=0, keepdims=True),
             jnp.sum(em * em, axis=0, keepdims=True)], axis=0)

        @pl.when(i == 0)
        def _():
            sum_ref[...] = jnp.zeros_like(sum_ref)

        sum_ref[...] += both

    return pl.pallas_call(
        body,
        grid=(NBLK,),
        in_specs=[pl.BlockSpec((BN, FW), lambda i: (i, 0)),
                  pl.BlockSpec((BN, FW), lambda i: (i, 0)),
                  pl.BlockSpec((2, BN, DEGW), lambda i: (0, i, 0)),
                  pl.BlockSpec((1, FW), lambda i: (0, 0))],
        out_specs=[pl.BlockSpec((BN, FW), lambda i: (i, 0)),
                   pl.BlockSpec((2, FW), lambda i: (0, 0))],
        out_shape=[jax.ShapeDtypeStruct((NPAD, FW), jnp.float32),
                   jax.ShapeDtypeStruct((2, FW), jnp.float32)],
    )(sseg, yc, degp, br)


def _gnorm_cols(hc, m, ms, w, b, eh2):
    var = eh2 - m * m * ms * (2.0 - ms)
    return (hc - m * ms) * (w * lax.rsqrt(var + 1e-5)) + b


def _mm2(h1, sums, degp, gw, gb, gms, W2):
    def body(h_ref, sm_ref, d_ref, gw_ref, gb_ref, gms_ref, w_ref, o_ref):
        dinv = _dinv_from(d_ref[...])
        sm = sm_ref[...]
        m = sm[0:1] * (1.0 / N)
        eh2 = sm[1:2] * (1.0 / N)
        g = _gnorm_cols(h_ref[...], m, gms_ref[...], gw_ref[...],
                        gb_ref[...], eh2)
        o_ref[...] = dinv * jnp.dot(g, w_ref[...],
                                    preferred_element_type=jnp.float32)

    return pl.pallas_call(
        body,
        grid=(NBLK,),
        in_specs=[pl.BlockSpec((BN, 128), lambda i: (i, 0)),
                  pl.BlockSpec((2, 128), lambda i: (0, 0)),
                  pl.BlockSpec((2, BN, DEGW), lambda i: (0, i, 0)),
                  pl.BlockSpec((1, 128), lambda i: (0, 0)),
                  pl.BlockSpec((1, 128), lambda i: (0, 0)),
                  pl.BlockSpec((1, 128), lambda i: (0, 0)),
                  pl.BlockSpec((128, 64), lambda i: (0, 0))],
        out_specs=pl.BlockSpec((BN, 64), lambda i: (i, 0)),
        out_shape=jax.ShapeDtypeStruct((NPAD, 64), jnp.float32),
    )(h1, sums, degp, gw, gb, gms, W2)


def _norm(h2, sums, gw, gb, gms, fw):
    def body(h_ref, sm_ref, gw_ref, gb_ref, gms_ref, o_ref):
        sm = sm_ref[...]
        m = sm[0:1] * (1.0 / N)
        eh2 = sm[1:2] * (1.0 / N)
        o_ref[...] = _gnorm_cols(h_ref[...], m, gms_ref[...], gw_ref[...],
                                 gb_ref[...], eh2)

    return pl.pallas_call(
        body,
        grid=(NBLK,),
        in_specs=[pl.BlockSpec((BN, fw), lambda i: (i, 0)),
                  pl.BlockSpec((2, fw), lambda i: (0, 0)),
                  pl.BlockSpec((1, fw), lambda i: (0, 0)),
                  pl.BlockSpec((1, fw), lambda i: (0, 0)),
                  pl.BlockSpec((1, fw), lambda i: (0, 0))],
        out_specs=pl.BlockSpec((BN, fw), lambda i: (i, 0)),
        out_shape=jax.ShapeDtypeStruct((NPAD, fw), jnp.float32),
    )(h2, sums, gw, gb, gms)


def _head(A, W1h, b1r, bng, bnb, bnm, bnv, W2h, b2r):
    KB = 2432
    KT = 14592 // KB

    def body(a_ref, w_ref, b1_ref, g_ref, bb_ref, m_ref, v_ref,
             w2_ref, b2_ref, o_ref, acc_ref):
        k = pl.program_id(0)

        @pl.when(k == 0)
        def _():
            acc_ref[...] = jnp.zeros_like(acc_ref)

        acc_ref[...] += jnp.dot(a_ref[...], w_ref[...],
                                preferred_element_type=jnp.float32)

        @pl.when(k == KT - 1)
        def _():
            z = acc_ref[...] + b1_ref[...]
            z = jnp.where(z > 0, z, jnp.exp(z) - 1.0)
            z = (z - m_ref[...]) * (g_ref[...] * lax.rsqrt(v_ref[...] + 1e-5)) \
                + bb_ref[...]
            o_ref[...] = jnp.dot(z, w2_ref[...],
                                 preferred_element_type=jnp.float32) \
                + b2_ref[...]

    return pl.pallas_call(
        body,
        grid=(KT,),
        in_specs=[pl.BlockSpec((150, KB), lambda k: (0, k)),
                  pl.BlockSpec((KB, 128), lambda k: (k, 0)),
                  pl.BlockSpec((1, 128), lambda k: (0, 0)),
                  pl.BlockSpec((1, 128), lambda k: (0, 0)),
                  pl.BlockSpec((1, 128), lambda k: (0, 0)),
                  pl.BlockSpec((1, 128), lambda k: (0, 0)),
                  pl.BlockSpec((1, 128), lambda k: (0, 0)),
                  pl.BlockSpec((128, 10), lambda k: (0, 0)),
                  pl.BlockSpec((1, 10), lambda k: (0, 0))],
        out_specs=pl.BlockSpec((150, 10), lambda k: (0, 0)),
        out_shape=jax.ShapeDtypeStruct((150, 10), jnp.float32),
        scratch_shapes=[pltpu.VMEM((150, 128), jnp.float32)],
    )(A, W1h, b1r, bng, bnb, bnm, bnv, W2h, b2r)


def kernel(x, edge_index, W1, b1, W2, b2, gn1_w, gn1_b, gn1_ms,
           gn2_w, gn2_b, gn2_ms, lin1_W, lin1_b, bn_g, bn_b, bn_m, bn_v,
           lin2_W, lin2_b):
    src = edge_index[0].astype(jnp.int32)
    dst = edge_index[1].astype(jnp.int32)
    pad_e = EPAD - E
    pad_dst = N + jnp.arange(pad_e, dtype=jnp.int32) % (NPAD - N)
    srcr = jnp.concatenate([src, jnp.zeros((pad_e,), jnp.int32)])
    dstr = jnp.concatenate([dst, pad_dst])
    xp = jnp.pad(x, ((0, NPAD - N), (0, 0)))
    ones16 = jnp.ones((128, DEGW), jnp.float32)
    zer16 = jnp.zeros((TPN, DEGW), jnp.float32)
    zerc = jnp.zeros((TPN, CW), jnp.float32)

    srcr2 = srcr.reshape(NROWS, 128)
    dstr2 = dstr.reshape(NROWS, 128)

    degp = _deg_sc(dstr, ones16, zer16)
    y1 = _mm1y(xp, W1, degp)
    s1 = _scat4(y1.reshape(NPAD * NCH1, CW), srcr2, dstr2, zerc)
    h1, sums1 = _stat(s1, y1, degp, b1.reshape(1, 128), NCH1)
    y2 = _mm2(h1, sums1, degp, gn1_w.reshape(1, 128),
              gn1_b.reshape(1, 128), gn1_ms.reshape(1, 128), W2)
    s2 = _scat2(y2.reshape(NPAD * NCH2, CW), srcr2, dstr2, zerc)
    h2, sums2 = _stat(s2, y2, degp, b2.reshape(1, 64), NCH2)
    g2 = _norm(h2, sums2, gn2_w.reshape(1, 64), gn2_b.reshape(1, 64),
               gn2_ms.reshape(1, 64), 64)
    A = g2[:N].reshape(150, 228 * 64)
    return _head(A, lin1_W, lin1_b.reshape(1, 128), bn_g.reshape(1, 128),
                 bn_b.reshape(1, 128), bn_m.reshape(1, 128),
                 bn_v.reshape(1, 128), lin2_W, lin2_b.reshape(1, 10))
